# baseline (speedup 1.0000x reference)
import functools

import jax
import jax.numpy as jnp
import numpy as np
from jax import lax
from jax.experimental import pallas as pl
from jax.experimental.pallas import tpu as pltpu
from jax.experimental.pallas import tpu_sc as plsc

_D = 4096
_E = 64
_K = 8
_S = 2
_BLOCK = 1024
_I32MIN = np.int32(-(2**31))
_MASK7F = np.int32(0x7FFFFFFF)
_NC = 2
_NS = 16
_ROWS_PER_W = 16384 // (_NC * _NS)
_CHUNK = 128


def _matmul_kernel(x_ref, w_ref, b_ref, logits_ref):
    logits_ref[...] = (
        jnp.dot(x_ref[...], w_ref[...], preferred_element_type=jnp.float32)
        + b_ref[...]
    )


def _tc_logits(inputs, W, b):
    n = inputs.shape[0]
    return pl.pallas_call(
        _matmul_kernel,
        grid=(n // _BLOCK,),
        in_specs=[
            pl.BlockSpec((_BLOCK, _D), lambda i: (i, 0)),
            pl.BlockSpec((_D, _E), lambda i: (0, 0)),
            pl.BlockSpec((1, _E), lambda i: (0, 0)),
        ],
        out_specs=pl.BlockSpec((_BLOCK, _E), lambda i: (i, 0)),
        out_shape=jax.ShapeDtypeStruct((n, _E), jnp.float32),
        compiler_params=pltpu.CompilerParams(
            dimension_semantics=("parallel",),
        ),
    )(inputs, W, b.reshape(1, _E))


def _sc_route_kernel(logits_hbm, probs_hbm, idx_hbm, lg_v, p_v, i_v):
    wid = lax.axis_index("s") * _NC + lax.axis_index("c")
    base = wid * _ROWS_PER_W

    lane = lax.iota(jnp.int32, 16)

    lane_f = lane.astype(jnp.float32)

    def _bfly(x, op):
        for s in (8, 4, 2, 1):
            x = op(x, x.at[lane ^ np.int32(s)].get(mode="promise_in_bounds"))
        return x

    def row_body(r, _):
        vs = [lg_v[r, pl.ds(g * 16, 16)] for g in range(4)]
        shared0 = vs[0][0]
        shared1 = vs[0][1]
        gidx = [lane_f + np.float32(g * 16) for g in range(4)]
        vs[0] = jnp.where(lane >= _S, vs[0], np.float32(-jnp.inf))

        vvec = jnp.full((16,), np.float32(-jnp.inf), jnp.float32)
        ivec_f = jnp.zeros((16,), jnp.float32)
        for k in range(_K):
            m = _bfly(jnp.maximum(jnp.maximum(vs[0], vs[1]),
                                  jnp.maximum(vs[2], vs[3])), jnp.maximum)
            cands = [jnp.where(vs[g] == m, gidx[g], np.float32(_E))
                     for g in range(4)]
            im = _bfly(jnp.minimum(jnp.minimum(cands[0], cands[1]),
                                   jnp.minimum(cands[2], cands[3])), jnp.minimum)
            sel = lane == np.int32(_S + k)
            vvec = jnp.where(sel, m, vvec)
            ivec_f = jnp.where(sel, im, ivec_f)
            for g in range(4):
                vs[g] = jnp.where(gidx[g] == im, np.float32(-jnp.inf), vs[g])

        vvec = jnp.where(lane == 0, shared0,
                         jnp.where(lane == 1, shared1, vvec))
        ivec_f = jnp.where(lane == 1, np.float32(1.0), ivec_f)
        live = lane < np.int32(_S + _K)
        vvec = jnp.where(live, vvec, np.float32(-1e30))
        e = jnp.exp(vvec - _bfly(vvec, jnp.maximum))
        p = e / _bfly(e, jnp.add)
        p_v[r, :] = p
        i_v[r, :] = jnp.where(live, ivec_f.astype(jnp.int32), 0)
        return ()

    for c in range(_ROWS_PER_W // _CHUNK):
        cbase = base + c * _CHUNK
        pltpu.sync_copy(logits_hbm.at[pl.ds(cbase, _CHUNK)], lg_v)
        lax.fori_loop(0, _CHUNK, row_body, ())
        pltpu.sync_copy(p_v, probs_hbm.at[pl.ds(cbase, _CHUNK)])
        pltpu.sync_copy(i_v, idx_hbm.at[pl.ds(cbase, _CHUNK)])


@functools.partial(
    pl.kernel,
    mesh=plsc.VectorSubcoreMesh(core_axis_name="c", subcore_axis_name="s"),
    out_type=[
        jax.ShapeDtypeStruct((16384, 16), jnp.float32),
        jax.ShapeDtypeStruct((16384, 16), jnp.int32),
    ],
    scratch_types=[
        pltpu.VMEM((_CHUNK, _E), jnp.float32),
        pltpu.VMEM((_CHUNK, 16), jnp.float32),
        pltpu.VMEM((_CHUNK, 16), jnp.int32),
    ],
)
def _sc_route(logits_hbm, probs_hbm, idx_hbm, lg_v, p_v, i_v):
    _sc_route_kernel(logits_hbm, probs_hbm, idx_hbm, lg_v, p_v, i_v)


def kernel(inputs, W, b):
    logits = _tc_logits(inputs, W, b)
    probs16, idx16 = _sc_route(logits)
    return probs16[:, : _S + _K], idx16[:, : _S + _K], logits


# speedup vs baseline: 1.4966x; 1.4966x over previous
import jax
import jax.numpy as jnp
import numpy as np
from jax.experimental import pallas as pl
from jax.experimental.pallas import tpu as pltpu

_D = 4096
_E = 64
_K = 8
_S = 2
_BLOCK = 1024
_I32MIN = np.int32(-(2**31))
_MASK7F = np.int32(0x7FFFFFFF)


def _gate_kernel(x_ref, w_ref, b_ref, probs_ref, idx_ref, logits_ref):
    logits = jnp.dot(x_ref[...], w_ref[...], preferred_element_type=jnp.float32)
    logits = logits + b_ref[...]
    logits_ref[...] = logits

    tr = logits.T
    bsz = tr.shape[1]
    bits = jax.lax.bitcast_convert_type(tr, jnp.int32)
    skey = bits ^ ((bits >> 31) & _MASK7F)
    row = jax.lax.broadcasted_iota(jnp.int32, tr.shape, 0)
    packed = (skey & np.int32(-64)) | (np.int32(_E - 1) - row)
    work = jnp.where(row >= _S, packed, _I32MIN)

    ms = []
    for _ in range(_K):
        m = jnp.max(work, axis=0, keepdims=True)
        ms.append(m)
        work = jnp.where(work == m, _I32MIN, work)
    mstack = jnp.concatenate(ms, axis=0)

    spec_idx = np.int32(_E - 1) - (mstack & np.int32(_E - 1))
    sv = mstack & np.int32(-64)
    spec_vals = jax.lax.bitcast_convert_type(sv ^ ((sv >> 31) & _MASK7F),
                                             jnp.float32)

    tv = jnp.concatenate([tr[:_S, :], spec_vals], axis=0)
    shared_idx = jax.lax.broadcasted_iota(jnp.int32, (_S, bsz), 0)
    ti = jnp.concatenate([shared_idx, spec_idx], axis=0)

    mx = jnp.max(tv, axis=0, keepdims=True)
    e = jnp.exp(tv - mx)
    p = e / jnp.sum(e, axis=0, keepdims=True)

    pad = jnp.zeros((16 - _S - _K, bsz), jnp.float32)
    probs_ref[...] = jnp.concatenate([p, pad], axis=0).T[:, : _S + _K]
    ipad = jnp.zeros((16 - _S - _K, bsz), jnp.int32)
    idx_ref[...] = jnp.concatenate([ti, ipad], axis=0).T[:, : _S + _K]


def kernel(inputs, W, b):
    n = inputs.shape[0]
    grid = (n // _BLOCK,)
    probs, idx, logits = pl.pallas_call(
        _gate_kernel,
        grid=grid,
        in_specs=[
            pl.BlockSpec((_BLOCK, _D), lambda i: (i, 0)),
            pl.BlockSpec((_D, _E), lambda i: (0, 0)),
            pl.BlockSpec((1, _E), lambda i: (0, 0)),
        ],
        out_specs=[
            pl.BlockSpec((_BLOCK, _S + _K), lambda i: (i, 0)),
            pl.BlockSpec((_BLOCK, _S + _K), lambda i: (i, 0)),
            pl.BlockSpec((_BLOCK, _E), lambda i: (i, 0)),
        ],
        out_shape=[
            jax.ShapeDtypeStruct((n, _S + _K), jnp.float32),
            jax.ShapeDtypeStruct((n, _S + _K), jnp.int32),
            jax.ShapeDtypeStruct((n, _E), jnp.float32),
        ],
        compiler_params=pltpu.CompilerParams(
            dimension_semantics=("parallel",),
        ),
    )(inputs, W, b.reshape(1, _E))
    return probs, idx, logits


# speedup vs baseline: 1.4995x; 1.0020x over previous
import jax
import jax.numpy as jnp
import numpy as np
from jax.experimental import pallas as pl
from jax.experimental.pallas import tpu as pltpu

_D = 4096
_E = 64
_K = 8
_S = 2
_BLOCK = 1024
_I32MIN = np.int32(-(2**31))
_MASK7F = np.int32(0x7FFFFFFF)


def _gate_kernel(x_ref, w_ref, b_ref, probs_ref, idx_ref, logits_ref):
    logits = jnp.dot(x_ref[...], w_ref[...], preferred_element_type=jnp.float32)
    logits = logits + b_ref[...]
    logits_ref[...] = logits

    tr = logits.T
    bsz = tr.shape[1]
    rowf = jax.lax.broadcasted_iota(jnp.int32, tr.shape, 0).astype(jnp.float32)
    work = jnp.where(rowf >= _S, tr, -jnp.inf)
    vals, idxs = [], []
    for _ in range(_K):
        m = jnp.max(work, axis=0, keepdims=True)
        im = jnp.min(jnp.where(work == m, rowf, np.float32(_E)),
                     axis=0, keepdims=True)
        vals.append(m)
        idxs.append(im)
        work = jnp.where(rowf == im, -jnp.inf, work)

    spec_idx = jnp.concatenate(idxs, axis=0).astype(jnp.int32)
    tv = jnp.concatenate([tr[:_S, :]] + vals, axis=0)
    shared_idx = jax.lax.broadcasted_iota(jnp.int32, (_S, bsz), 0)
    ti = jnp.concatenate([shared_idx, spec_idx], axis=0)

    mx = jnp.max(tv, axis=0, keepdims=True)
    e = jnp.exp(tv - mx)
    p = e / jnp.sum(e, axis=0, keepdims=True)

    pad = jnp.zeros((16 - _S - _K, bsz), jnp.float32)
    probs_ref[...] = jnp.concatenate([p, pad], axis=0).T[:, : _S + _K]
    ipad = jnp.zeros((16 - _S - _K, bsz), jnp.int32)
    idx_ref[...] = jnp.concatenate([ti, ipad], axis=0).T[:, : _S + _K]


def kernel(inputs, W, b):
    n = inputs.shape[0]
    grid = (n // _BLOCK,)
    probs, idx, logits = pl.pallas_call(
        _gate_kernel,
        grid=grid,
        in_specs=[
            pl.BlockSpec((_BLOCK, _D), lambda i: (i, 0)),
            pl.BlockSpec((_D, _E), lambda i: (0, 0)),
            pl.BlockSpec((1, _E), lambda i: (0, 0)),
        ],
        out_specs=[
            pl.BlockSpec((_BLOCK, _S + _K), lambda i: (i, 0)),
            pl.BlockSpec((_BLOCK, _S + _K), lambda i: (i, 0)),
            pl.BlockSpec((_BLOCK, _E), lambda i: (i, 0)),
        ],
        out_shape=[
            jax.ShapeDtypeStruct((n, _S + _K), jnp.float32),
            jax.ShapeDtypeStruct((n, _S + _K), jnp.int32),
            jax.ShapeDtypeStruct((n, _E), jnp.float32),
        ],
        compiler_params=pltpu.CompilerParams(
            dimension_semantics=("parallel",),
        ),
    )(inputs, W, b.reshape(1, _E))
    return probs, idx, logits
